# baseline (speedup 1.0000x reference)
import jax
import jax.numpy as jnp
from jax import lax
from jax.experimental import pallas as pl
from jax.experimental.pallas import tpu as pltpu
from jax.experimental.pallas import tpu_sc as plsc

B, P, T, D = 16, 8, 2048, 128
N = B * P * T
DW = D // 2
DRUMS_ID = 128
TRACKS = B * P
NC, NS = 2, 16
NW = NC * NS
TPW = TRACKS // NW
CHUNK = 256
NCH = T // CHUNK
CPW = TPW * NCH
GPC = CHUNK // 16
OUTW = CHUNK * DW


def _pack_rows(w):
    v = w.shape[0]
    return lax.bitcast_convert_type(
        w.astype(jnp.bfloat16).reshape(v, DW, 2), jnp.int32).reshape(-1)


def _sc_embed_body(gmdr, velpk, instW, pv, off16, pids, e_out, einst,
                   gmdr_v, velpk_v, instrow_v, pid_v, off_v,
                   pv_v, out_v, sem_i, sem_o0, sem_o1):
    wid = lax.axis_index("s") * NC + lax.axis_index("c")
    pltpu.sync_copy(gmdr, gmdr_v)
    pltpu.sync_copy(velpk, velpk_v)
    pltpu.sync_copy(pids, pid_v)
    pltpu.sync_copy(off16, off_v)
    pltpu.async_copy(instW.at[pid_v], instrow_v, sem_i).wait()
    for trk in range(TPW):
        track = wid * TPW + trk
        pltpu.sync_copy(instrow_v.at[pl.ds(track, 1)],
                        einst.at[pl.ds(track, 1)])

    iota64 = lax.iota(jnp.int32, 16) * DW
    sems = [sem_o0, sem_o1]

    def chunk_body(ci, carry):
        track = wid * TPW + ci // NCH
        base = (wid * TPW) * T + ci * CHUNK
        off_vec = off_v[pl.ds(track * 16, 16)]
        pltpu.sync_copy(pv.at[wid * CPW + ci], pv_v)

        for buf in (0, 1):

            @pl.when(ci % 2 == buf)
            def _do_chunk():
                obeg = buf * OUTW

                @pl.when(ci >= 2)
                def _wait_prev():
                    pltpu.make_async_copy(
                        out_v.at[pl.ds(obeg, OUTW)],
                        e_out.at[pl.ds(base * DW, OUTW)],
                        sems[buf]).wait()

                def per_group(r, carry):
                    pbase = (pv_v[0, pl.ds(r * 16, 16)] + off_vec) * DW
                    vbase = pv_v[1, pl.ds(r * 16, 16)] * DW
                    obase = iota64 + (obeg + r * 16 * DW)
                    for c in range(DW):
                        g1 = plsc.load_gather(gmdr_v, [pbase + c])
                        g2 = plsc.load_gather(velpk_v, [vbase + c])
                        s = plsc.bitcast(g1, jnp.bfloat16) + \
                            plsc.bitcast(g2, jnp.bfloat16)
                        plsc.store_scatter(out_v, [obase + c],
                                           plsc.bitcast(s, jnp.int32))
                    return carry

                lax.fori_loop(0, GPC, per_group, 0)
                pltpu.async_copy(
                    out_v.at[pl.ds(obeg, OUTW)],
                    e_out.at[pl.ds(base * DW, OUTW)],
                    sems[buf])

        return carry

    lax.fori_loop(0, CPW, chunk_body, 0)
    for buf in (0, 1):
        pltpu.make_async_copy(
            out_v.at[pl.ds(buf * OUTW, OUTW)],
            e_out.at[pl.ds(0, OUTW)],
            sems[buf]).wait()


def _make_sc_embed():
    return pl.kernel(
        _sc_embed_body,
        out_type=(
            jax.ShapeDtypeStruct((N * DW,), jnp.int32),
            jax.ShapeDtypeStruct((TRACKS, D), jnp.float32),
        ),
        mesh=plsc.VectorSubcoreMesh(core_axis_name="c", subcore_axis_name="s"),
        compiler_params=pltpu.CompilerParams(needs_layout_passes=False),
        scratch_types=[
            pltpu.VMEM((1024 * DW,), jnp.int32),
            pltpu.VMEM((128 * DW,), jnp.int32),
            pltpu.VMEM((TRACKS, D), jnp.float32),
            pltpu.VMEM((TRACKS,), jnp.int32),
            pltpu.VMEM((TRACKS * 16,), jnp.int32),
            pltpu.VMEM((2, CHUNK), jnp.int32),
            pltpu.VMEM((2 * OUTW,), jnp.int32),
            pltpu.SemaphoreType.DMA,
            pltpu.SemaphoreType.DMA,
            pltpu.SemaphoreType.DMA,
        ],
    )


BLK = T


def _tc_body(e_ref, dur_ref, mask_ref, inst_ref, w1_ref, b1_ref, w2_ref,
             b2_ref, out_ref):
    d = jnp.log(1.0 + dur_ref[...])
    h = d * w1_ref[...] + b1_ref[...]
    h = h * (1.0 / (1.0 + jnp.exp(-h)))
    h2 = lax.dot_general(h, w2_ref[...], (((1,), (1,)), ((), ())),
                         preferred_element_type=jnp.float32)
    e = e_ref[...].astype(jnp.float32)
    inst = inst_ref[...].reshape(1, D)
    out_ref[...] = (e + h2 + b2_ref[...] + inst) * mask_ref[...]


def kernel(program_ids, pitch_tokens, velocity_tokens, note_durations_beats,
           attention_mask, gm_W, dr_W, vel_W, inst_W, W1, b1, W2, b2):
    pids = program_ids.reshape(-1).astype(jnp.int32)
    gmdr = _pack_rows(jnp.concatenate([gm_W, dr_W], axis=0))
    velpk = _pack_rows(vel_W)
    off = jnp.where(pids == DRUMS_ID, 512, 0).astype(jnp.int32)
    off16 = jnp.broadcast_to(off[:, None], (TRACKS, 16)).reshape(-1)
    pitch = pitch_tokens.reshape(-1).astype(jnp.int32)
    vel = velocity_tokens.reshape(-1).astype(jnp.int32)
    pv = jnp.stack([pitch.reshape(N // CHUNK, CHUNK),
                    vel.reshape(N // CHUNK, CHUNK)], axis=1)

    e_i32, einst = _make_sc_embed()(gmdr, velpk, inst_W, pv, off16, pids)
    e_bf16 = lax.bitcast_convert_type(
        e_i32.reshape(N, DW), jnp.bfloat16).reshape(N, D)

    dur2d = note_durations_beats.reshape(N, 1)
    mask2d = attention_mask.reshape(N, 1).astype(jnp.float32)
    w1row = W1.reshape(1, D)
    b1row = b1.reshape(1, D)
    b2row = b2.reshape(1, D)

    out2d = pl.pallas_call(
        _tc_body,
        grid=(N // BLK,),
        in_specs=[
            pl.BlockSpec((BLK, D), lambda i: (i, 0)),
            pl.BlockSpec((BLK, 1), lambda i: (i, 0)),
            pl.BlockSpec((BLK, 1), lambda i: (i, 0)),
            pl.BlockSpec((1, 1, D), lambda i: (i, 0, 0)),
            pl.BlockSpec((1, D), lambda i: (0, 0)),
            pl.BlockSpec((1, D), lambda i: (0, 0)),
            pl.BlockSpec((D, D), lambda i: (0, 0)),
            pl.BlockSpec((1, D), lambda i: (0, 0)),
        ],
        out_specs=pl.BlockSpec((BLK, D), lambda i: (i, 0)),
        out_shape=jax.ShapeDtypeStruct((N, D), jnp.float32),
    )(e_bf16, dur2d, mask2d, einst.reshape(TRACKS, 1, D), w1row, b1row, W2,
      b2row)

    return out2d.reshape(B, P, T, D)


# speedup vs baseline: 1.1729x; 1.1729x over previous
import jax
import jax.numpy as jnp
from jax import lax
from jax.experimental import pallas as pl
from jax.experimental.pallas import tpu as pltpu
from jax.experimental.pallas import tpu_sc as plsc

B, P, T, D = 16, 8, 2048, 128
N = B * P * T
DW = D // 2
DRUMS_ID = 128
TRACKS = B * P
NC, NS = 2, 16
NW = NC * NS
TPW = TRACKS // NW
CHUNK = 256
NCH = T // CHUNK
CPW = TPW * NCH
GPC = CHUNK // 16
OUTW = CHUNK * DW


def _pack_rows(w):
    v = w.shape[0]
    return lax.bitcast_convert_type(
        w.astype(jnp.bfloat16).reshape(v, DW, 2), jnp.int32).reshape(-1)


def _sc_embed_body(gmdr, velpk, instW, pv, off16, pids, e_out, einst,
                   gmdr_v, velpk_v, instrow_v, pid_v, off_v,
                   pv_v, out_v, sem_i, sem_o0, sem_o1):
    wid = lax.axis_index("s") * NC + lax.axis_index("c")
    pltpu.sync_copy(gmdr, gmdr_v)
    pltpu.sync_copy(velpk, velpk_v)
    pltpu.sync_copy(pids, pid_v)
    pltpu.sync_copy(off16, off_v)
    pltpu.async_copy(instW.at[pid_v], instrow_v, sem_i).wait()
    for trk in range(TPW):
        track = wid * TPW + trk
        pltpu.sync_copy(instrow_v.at[pl.ds(track, 1)],
                        einst.at[pl.ds(track, 1)])

    iota64 = lax.iota(jnp.int32, 16) * DW
    sems = [sem_o0, sem_o1]

    def chunk_body(ci, carry):
        track = wid * TPW + ci // NCH
        base = (wid * TPW) * T + ci * CHUNK
        off_vec = off_v[pl.ds(track * 16, 16)]
        pltpu.sync_copy(pv.at[wid * CPW + ci], pv_v)

        for buf in (0, 1):

            @pl.when(ci % 2 == buf)
            def _do_chunk():
                obeg = buf * OUTW

                @pl.when(ci >= 2)
                def _wait_prev():
                    pltpu.make_async_copy(
                        out_v.at[pl.ds(obeg, OUTW)],
                        e_out.at[pl.ds(base * DW, OUTW)],
                        sems[buf]).wait()

                @plsc.parallel_loop(0, GPC)
                def per_group(r):
                    pbase = (pv_v[0, pl.ds(r * 16, 16)] + off_vec) * DW
                    vbase = pv_v[1, pl.ds(r * 16, 16)] * DW
                    obase = iota64 + (obeg + r * 16 * DW)
                    for c in range(DW):
                        g1 = plsc.load_gather(gmdr_v, [pbase + c])
                        g2 = plsc.load_gather(velpk_v, [vbase + c])
                        s = plsc.bitcast(g1, jnp.bfloat16) + \
                            plsc.bitcast(g2, jnp.bfloat16)
                        plsc.store_scatter(out_v, [obase + c],
                                           plsc.bitcast(s, jnp.int32))
                pltpu.async_copy(
                    out_v.at[pl.ds(obeg, OUTW)],
                    e_out.at[pl.ds(base * DW, OUTW)],
                    sems[buf])

        return carry

    lax.fori_loop(0, CPW, chunk_body, 0)
    for buf in (0, 1):
        pltpu.make_async_copy(
            out_v.at[pl.ds(buf * OUTW, OUTW)],
            e_out.at[pl.ds(0, OUTW)],
            sems[buf]).wait()


def _make_sc_embed():
    return pl.kernel(
        _sc_embed_body,
        out_type=(
            jax.ShapeDtypeStruct((N * DW,), jnp.int32),
            jax.ShapeDtypeStruct((TRACKS, D), jnp.float32),
        ),
        mesh=plsc.VectorSubcoreMesh(core_axis_name="c", subcore_axis_name="s"),
        compiler_params=pltpu.CompilerParams(needs_layout_passes=False),
        scratch_types=[
            pltpu.VMEM((1024 * DW,), jnp.int32),
            pltpu.VMEM((128 * DW,), jnp.int32),
            pltpu.VMEM((TRACKS, D), jnp.float32),
            pltpu.VMEM((TRACKS,), jnp.int32),
            pltpu.VMEM((TRACKS * 16,), jnp.int32),
            pltpu.VMEM((2, CHUNK), jnp.int32),
            pltpu.VMEM((2 * OUTW,), jnp.int32),
            pltpu.SemaphoreType.DMA,
            pltpu.SemaphoreType.DMA,
            pltpu.SemaphoreType.DMA,
        ],
    )


BLK = T


def _tc_body(e_ref, dur_ref, mask_ref, inst_ref, w1_ref, b1_ref, w2_ref,
             out_ref):
    d = jnp.log(1.0 + dur_ref[...])
    h = d * w1_ref[...] + b1_ref[...]
    h = h * (1.0 / (1.0 + jnp.exp(-h)))
    h2 = lax.dot_general(h, w2_ref[...], (((1,), (1,)), ((), ())),
                         preferred_element_type=jnp.float32)
    e = e_ref[...].astype(jnp.float32)
    inst = inst_ref[...].reshape(1, D)
    out_ref[...] = (e + h2 + inst) * mask_ref[...]


def kernel(program_ids, pitch_tokens, velocity_tokens, note_durations_beats,
           attention_mask, gm_W, dr_W, vel_W, inst_W, W1, b1, W2, b2):
    pids = program_ids.reshape(-1).astype(jnp.int32)
    gmdr = _pack_rows(jnp.concatenate([gm_W, dr_W], axis=0))
    velpk = _pack_rows(vel_W)
    off = jnp.where(pids == DRUMS_ID, 512, 0).astype(jnp.int32)
    off16 = jnp.broadcast_to(off[:, None], (TRACKS, 16)).reshape(-1)
    pitch = pitch_tokens.reshape(-1).astype(jnp.int32)
    vel = velocity_tokens.reshape(-1).astype(jnp.int32)
    pv = jnp.stack([pitch.reshape(N // CHUNK, CHUNK),
                    vel.reshape(N // CHUNK, CHUNK)], axis=1)

    e_i32, einst = _make_sc_embed()(gmdr, velpk, inst_W, pv, off16, pids)
    e_bf16 = lax.bitcast_convert_type(
        e_i32.reshape(N, DW), jnp.bfloat16).reshape(N, D)

    dur2d = note_durations_beats.reshape(N, 1)
    mask2d = attention_mask.reshape(N, 1).astype(jnp.float32)
    w1row = W1.reshape(1, D)
    b1row = b1.reshape(1, D)
    b2row = b2.reshape(1, D)

    out2d = pl.pallas_call(
        _tc_body,
        grid=(N // BLK,),
        in_specs=[
            pl.BlockSpec((BLK, D), lambda i: (i, 0)),
            pl.BlockSpec((BLK, 1), lambda i: (i, 0)),
            pl.BlockSpec((BLK, 1), lambda i: (i, 0)),
            pl.BlockSpec((1, 1, D), lambda i: (i, 0, 0)),
            pl.BlockSpec((1, D), lambda i: (0, 0)),
            pl.BlockSpec((1, D), lambda i: (0, 0)),
            pl.BlockSpec((D, D), lambda i: (0, 0)),
        ],
        out_specs=pl.BlockSpec((BLK, D), lambda i: (i, 0)),
        out_shape=jax.ShapeDtypeStruct((N, D), jnp.float32),
    )(e_bf16, dur2d, mask2d, (einst + b2row).reshape(TRACKS, 1, D), w1row,
      b1row, W2)

    return out2d.reshape(B, P, T, D)
